# Initial kernel scaffold; baseline (speedup 1.0000x reference)
import jax
import jax.numpy as jnp
from jax.experimental import pallas as pl


def kernel(xyz1, xyz2, points1, points2, W0, b0, g0, be0, W1, b1, g1, be1):
    raise NotImplementedError("write your pallas kernel here")


# speedup vs baseline: 18.7688x; 18.7688x over previous
import functools
import jax
import jax.numpy as jnp
from jax.experimental import pallas as pl

NB = 256


def _stage1_kernel(x1t_ref, x2t_ref, p1_ref, p2_ref, w0_ref, b0_ref,
                   h0_ref, s_ref, q_ref):
    b = pl.program_id(0)
    i = pl.program_id(1)
    x1 = x1t_ref[0]
    x2 = x2t_ref[0]
    M = x2.shape[1]
    cross = jax.lax.dot_general(x1, x2, (((0,), (0,)), ((), ())),
                                preferred_element_type=jnp.float32)
    n1 = jnp.sum(x1 * x1, axis=0)[:, None]
    n2 = jnp.sum(x2 * x2, axis=0)[None, :]
    dist = n1 + n2 - 2.0 * cross

    iota = jax.lax.broadcasted_iota(jnp.int32, dist.shape, 1)
    recip = 1.0 / (dist + 1e-8)
    d = dist
    wfull = jnp.zeros_like(dist)
    for _ in range(3):
        mval = jnp.min(d, axis=1, keepdims=True)
        first = jnp.min(jnp.where(d == mval, iota, M), axis=1, keepdims=True)
        sel = iota == first
        wfull = jnp.where(sel, recip, wfull)
        d = jnp.where(sel, jnp.inf, d)
    wfull = wfull / jnp.sum(wfull, axis=1, keepdims=True)

    interp = jax.lax.dot_general(wfull, p2_ref[0], (((1,), (0,)), ((), ())),
                                 preferred_element_type=jnp.float32)
    f = jnp.concatenate([p1_ref[0], interp], axis=1)
    h = jax.lax.dot_general(f, w0_ref[...], (((1,), (1,)), ((), ())),
                            preferred_element_type=jnp.float32)
    h = h + b0_ref[...]
    h0_ref[0] = h

    @pl.when(jnp.logical_and(b == 0, i == 0))
    def _():
        s_ref[...] = jnp.zeros_like(s_ref)
        q_ref[...] = jnp.zeros_like(q_ref)

    s_ref[...] += jnp.sum(h, axis=0, keepdims=True)
    q_ref[...] += jnp.sum(h * h, axis=0, keepdims=True)


def _stage2_kernel(count_inv, h0_ref, s_ref, q_ref, g_ref, be_ref, w1_ref,
                   b1_ref, h1_ref, s2_ref, q2_ref):
    b = pl.program_id(0)
    i = pl.program_id(1)
    mean = s_ref[...] * count_inv
    var = q_ref[...] * count_inv - mean * mean
    inv = jax.lax.rsqrt(var + 1e-5)
    scale = g_ref[...] * inv
    shift = be_ref[...] - mean * scale
    y = jnp.maximum(h0_ref[0] * scale + shift, 0.0)
    h = jax.lax.dot_general(y, w1_ref[...], (((1,), (1,)), ((), ())),
                            preferred_element_type=jnp.float32)
    h = h + b1_ref[...]
    h1_ref[0] = h

    @pl.when(jnp.logical_and(b == 0, i == 0))
    def _():
        s2_ref[...] = jnp.zeros_like(s2_ref)
        q2_ref[...] = jnp.zeros_like(q2_ref)

    s2_ref[...] += jnp.sum(h, axis=0, keepdims=True)
    q2_ref[...] += jnp.sum(h * h, axis=0, keepdims=True)


def _stage3_kernel(count_inv, h1_ref, s_ref, q_ref, g_ref, be_ref, out_ref):
    mean = s_ref[...] * count_inv
    var = q_ref[...] * count_inv - mean * mean
    inv = jax.lax.rsqrt(var + 1e-5)
    scale = g_ref[...] * inv
    shift = be_ref[...] - mean * scale
    out_ref[0] = jnp.maximum(h1_ref[0] * scale + shift, 0.0)


@jax.jit
def kernel(xyz1, xyz2, points1, points2, W0, b0, g0, be0, W1, b1, g1, be1):
    B, N, _ = xyz1.shape
    M = xyz2.shape[1]
    C1 = points1.shape[-1]
    C2 = points2.shape[-1]
    CH0 = W0.shape[0]
    CH1 = W1.shape[0]
    x1t = jnp.transpose(xyz1, (0, 2, 1))
    x2t = jnp.transpose(xyz2, (0, 2, 1))
    count_inv = 1.0 / float(B * N)

    grid = (B, N // NB)
    h0, s0, q0 = pl.pallas_call(
        _stage1_kernel,
        grid=grid,
        in_specs=[
            pl.BlockSpec((1, 3, NB), lambda b, i: (b, 0, i)),
            pl.BlockSpec((1, 3, M), lambda b, i: (b, 0, 0)),
            pl.BlockSpec((1, NB, C1), lambda b, i: (b, i, 0)),
            pl.BlockSpec((1, M, C2), lambda b, i: (b, 0, 0)),
            pl.BlockSpec((CH0, C1 + C2), lambda b, i: (0, 0)),
            pl.BlockSpec((1, CH0), lambda b, i: (0, 0)),
        ],
        out_specs=[
            pl.BlockSpec((1, NB, CH0), lambda b, i: (b, i, 0)),
            pl.BlockSpec((1, CH0), lambda b, i: (0, 0)),
            pl.BlockSpec((1, CH0), lambda b, i: (0, 0)),
        ],
        out_shape=[
            jax.ShapeDtypeStruct((B, N, CH0), jnp.float32),
            jax.ShapeDtypeStruct((1, CH0), jnp.float32),
            jax.ShapeDtypeStruct((1, CH0), jnp.float32),
        ],
    )(x1t, x2t, points1, points2, W0, b0.reshape(1, -1))

    h1, s1, q1 = pl.pallas_call(
        functools.partial(_stage2_kernel, count_inv),
        grid=grid,
        in_specs=[
            pl.BlockSpec((1, NB, CH0), lambda b, i: (b, i, 0)),
            pl.BlockSpec((1, CH0), lambda b, i: (0, 0)),
            pl.BlockSpec((1, CH0), lambda b, i: (0, 0)),
            pl.BlockSpec((1, CH0), lambda b, i: (0, 0)),
            pl.BlockSpec((1, CH0), lambda b, i: (0, 0)),
            pl.BlockSpec((CH1, CH0), lambda b, i: (0, 0)),
            pl.BlockSpec((1, CH1), lambda b, i: (0, 0)),
        ],
        out_specs=[
            pl.BlockSpec((1, NB, CH1), lambda b, i: (b, i, 0)),
            pl.BlockSpec((1, CH1), lambda b, i: (0, 0)),
            pl.BlockSpec((1, CH1), lambda b, i: (0, 0)),
        ],
        out_shape=[
            jax.ShapeDtypeStruct((B, N, CH1), jnp.float32),
            jax.ShapeDtypeStruct((1, CH1), jnp.float32),
            jax.ShapeDtypeStruct((1, CH1), jnp.float32),
        ],
    )(h0, s0, q0, g0.reshape(1, -1), be0.reshape(1, -1), W1,
      b1.reshape(1, -1))

    out = pl.pallas_call(
        functools.partial(_stage3_kernel, count_inv),
        grid=grid,
        in_specs=[
            pl.BlockSpec((1, NB, CH1), lambda b, i: (b, i, 0)),
            pl.BlockSpec((1, CH1), lambda b, i: (0, 0)),
            pl.BlockSpec((1, CH1), lambda b, i: (0, 0)),
            pl.BlockSpec((1, CH1), lambda b, i: (0, 0)),
            pl.BlockSpec((1, CH1), lambda b, i: (0, 0)),
        ],
        out_specs=pl.BlockSpec((1, NB, CH1), lambda b, i: (b, i, 0)),
        out_shape=jax.ShapeDtypeStruct((B, N, CH1), jnp.float32),
    )(h1, s1, q1, g1.reshape(1, -1), be1.reshape(1, -1))

    return out
